# baseline (speedup 1.0000x reference)
import functools

import jax
import jax.numpy as jnp
from jax import lax
from jax.experimental import pallas as pl
from jax.experimental.pallas import tpu as pltpu
from jax.experimental.pallas import tpu_sc as plsc

M, D, B = 1000000, 64, 16384
NC, NS = 2, 16
NW = NC * NS
BW = B // NW
BH = B // NS

_mesh = plsc.VectorSubcoreMesh(
    core_axis_name="c", subcore_axis_name="s", num_cores=NC, num_subcores=NS
)


@functools.partial(
    pl.kernel,
    out_type=jax.ShapeDtypeStruct((M,), jnp.int32),
    mesh=_mesh,
    compiler_params=pltpu.CompilerParams(use_tc_tiling_on_sc=False),
    scratch_types=[
        pltpu.VMEM((BW,), jnp.int32),
        pltpu.VMEM((BW,), jnp.int32),
    ],
)
def _k1_scatter_rep(idx_hbm, t_hbm, idx_v, pos_v):
    wid = lax.axis_index("s") * NC + lax.axis_index("c")
    base = wid * BW
    pltpu.sync_copy(idx_hbm.at[pl.ds(base, BW)], idx_v)
    for i in range(BW // 16):
        pos_v[pl.ds(i * 16, 16)] = base + i * 16 + lax.iota(jnp.int32, 16)
    pltpu.sync_copy(pos_v, t_hbm.at[idx_v])


@functools.partial(
    pl.kernel,
    out_type=jax.ShapeDtypeStruct((B,), jnp.int32),
    mesh=_mesh,
    compiler_params=pltpu.CompilerParams(use_tc_tiling_on_sc=False),
    scratch_types=[
        pltpu.VMEM((BW,), jnp.int32),
        pltpu.VMEM((BW,), jnp.int32),
    ],
)
def _k2a_gather_rep(t_hbm, idx_hbm, r_hbm, idx_v, r_v):
    wid = lax.axis_index("s") * NC + lax.axis_index("c")
    base = wid * BW
    pltpu.sync_copy(idx_hbm.at[pl.ds(base, BW)], idx_v)
    pltpu.sync_copy(t_hbm.at[idx_v], r_v)
    pltpu.sync_copy(r_v, r_hbm.at[pl.ds(base, BW)])


NWIN = (M // 128) // NW
STRIPE = NWIN * 128
TAILB = NW * STRIPE
DUMMY = B


@functools.partial(
    pl.kernel,
    out_type=jax.ShapeDtypeStruct((B + 16, 2 * D), jnp.float32),
    mesh=_mesh,
    compiler_params=pltpu.CompilerParams(needs_layout_passes=False),
    scratch_types=[
        pltpu.VMEM((B,), jnp.int32),
        pltpu.VMEM((STRIPE + 128,), jnp.int32),
        pltpu.VMEM((2 * D, 128), jnp.float32),
        pltpu.VMEM((16, 2 * D), jnp.float32),
        pltpu.VMEM((16,), jnp.int32),
        pltpu.SemaphoreType.DMA,
    ],
)
def _kscan(memt_hbm, t_hbm, idx_hbm, tailt_hbm, grep_hbm, idxf_v, ts_v, w2_v,
           stage_v, slot_v, sem):
    wid = lax.axis_index("s") * NC + lax.axis_index("c")
    pltpu.sync_copy(idx_hbm, idxf_v)
    pltpu.sync_copy(t_hbm.at[pl.ds(wid * STRIPE, STRIPE)],
                    ts_v.at[pl.ds(0, STRIPE)])

    @pl.when(wid < 4)
    def _():
        pltpu.sync_copy(t_hbm.at[pl.ds(TAILB + wid * 128, 128)],
                        ts_v.at[pl.ds(STRIPE, 128)])

    nwin = NWIN + jnp.where(wid < 4, 1, 0)

    def wbase(i):
        return jnp.where(i < NWIN, wid * STRIPE + i * 128, TAILB + wid * 128)

    def issue(i):
        par = lax.rem(i, 2)
        pltpu.async_copy(
            memt_hbm.at[:, pl.ds(pl.multiple_of(wbase(i), 128), 128)],
            w2_v.at[pl.ds(par * D, D)], sem)

    def process(kb, toff, ngroups, wref, rowoff):
        for g in range(ngroups):
            tv = ts_v[pl.ds(toff + g * 16, 16)]
            inb = (tv >= 0) & (tv < B)
            tcl = jnp.where(inb, tv, 0)
            iv = plsc.load_gather(idxf_v, [tcl])
            kv = kb + g * 16 + lax.iota(jnp.int32, 16)
            ok = inb & (iv == kv)
            cnt = jnp.sum(ok.astype(jnp.int32))

            @pl.when(cnt > 0)
            def _():
                slot_v[...] = jnp.where(ok, tv, DUMMY)
                for j in range(16):
                    cj = jnp.full((16,), g * 16 + j, jnp.int32)
                    for d0 in range(D // 16):
                        rv = rowoff + d0 * 16 + lax.iota(jnp.int32, 16)
                        stage_v[j, pl.ds(d0 * 16, 16)] = plsc.load_gather(
                            wref, [rv, cj])
                pltpu.sync_copy(stage_v, grep_hbm.at[slot_v])

    issue(0)

    def body(i, carry):
        pltpu.make_async_copy(
            memt_hbm.at[:, pl.ds(0, 128)], w2_v.at[pl.ds(0, D)], sem
        ).wait()

        @pl.when(i + 1 < nwin)
        def _():
            issue(i + 1)

        process(wbase(i), i * 128, 8, w2_v, lax.rem(i, 2) * D)
        return carry

    lax.fori_loop(0, nwin, body, 0)

    @pl.when(wid == 4)
    def _():
        pltpu.sync_copy(t_hbm.at[pl.ds(TAILB + 4 * 128, 64)],
                        ts_v.at[pl.ds(0, 64)])
        pltpu.sync_copy(tailt_hbm, w2_v.at[pl.ds(0, D)])
        process(TAILB + 4 * 128, 0, 4, w2_v, 0)


BHALF = B // NC
BZ = BHALF // NS


@functools.partial(
    pl.kernel,
    out_type=jax.ShapeDtypeStruct((B, D), jnp.float32),
    mesh=_mesh,
    compiler_params=pltpu.CompilerParams(use_tc_tiling_on_sc=False),
    scratch_types=[
        pltpu.VMEM((BH,), jnp.int32),
        pltpu.VMEM((BH,), jnp.int32),
        pltpu.VMEM((BH, D), jnp.float32),
        pltpu.VMEM_SHARED((BHALF + 16, D), jnp.float32),
    ],
)
def _k3_segment_sums(r_hbm, val_hbm, z_hbm, acc_hbm, r_v, rloc_v, val_v, acc_sp):
    c = lax.axis_index("c")
    s = lax.axis_index("s")
    base = s * BH
    zbase = s * BZ

    pltpu.sync_copy(z_hbm.at[pl.ds(0, BZ)], val_v.at[pl.ds(0, BZ)])
    pltpu.sync_copy(val_v.at[pl.ds(0, BZ)], acc_sp.at[pl.ds(zbase, BZ)])
    plsc.subcore_barrier()
    pltpu.sync_copy(r_hbm.at[pl.ds(base, BH)], r_v)
    pltpu.sync_copy(val_hbm.at[pl.ds(base, BH)], val_v)
    lo = c * BHALF
    for i in range(BH // 16):
        rv = r_v[pl.ds(i * 16, 16)] - lo
        ok = (rv >= 0) & (rv < BHALF)
        rloc_v[pl.ds(i * 16, 16)] = jnp.where(ok, rv, BHALF)
    pltpu.sync_copy(val_v, acc_sp.at[rloc_v], add=True)
    plsc.subcore_barrier()
    pltpu.sync_copy(acc_sp.at[pl.ds(zbase, BZ)], val_v.at[pl.ds(0, BZ)])
    pltpu.sync_copy(val_v.at[pl.ds(0, BZ)], acc_hbm.at[pl.ds(lo + zbase, BZ)])


@functools.partial(
    pl.kernel,
    out_type=(
        jax.ShapeDtypeStruct((B, D), jnp.float32),
        jax.ShapeDtypeStruct((B, 2 * D), jnp.float32),
    ),
    mesh=_mesh,
    compiler_params=pltpu.CompilerParams(use_tc_tiling_on_sc=False),
    scratch_types=[
        pltpu.VMEM((BW,), jnp.int32),
        pltpu.VMEM((BW, D), jnp.float32),
        pltpu.VMEM((BW, 2 * D), jnp.float32),
    ],
)
def _k4_gather_sums(acc_hbm, grep_hbm, r_hbm, s_hbm, g_hbm, r_v, s_v, g_v):
    wid = lax.axis_index("s") * NC + lax.axis_index("c")
    base = wid * BW
    pltpu.sync_copy(r_hbm.at[pl.ds(base, BW)], r_v)
    pltpu.sync_copy(acc_hbm.at[r_v], s_v)
    pltpu.sync_copy(s_v, s_hbm.at[pl.ds(base, BW)])
    pltpu.sync_copy(grep_hbm.at[r_v], g_v)
    pltpu.sync_copy(g_v, g_hbm.at[pl.ds(base, BW)])


def _k5_body(g_ref, s_ref, w_ref, b_ref, o_ref):
    x = g_ref[:, :D] + s_ref[...]
    o_ref[...] = (
        jnp.dot(x, w_ref[...], preferred_element_type=jnp.float32) + b_ref[...]
    )


_BLK = 2048


@jax.jit
def kernel(mem, idx, val, W, b):
    idx = idx.astype(jnp.int32)
    memt = mem.T
    z = jnp.zeros((BZ, D), jnp.float32)
    t = _k1_scatter_rep(idx)
    r = _k2a_gather_rep(t, idx)
    tailt = jnp.pad(
        lax.slice(mem, (NW * (M // 128 // NW) * 128 + 512, 0), (M, D)).T,
        ((0, 0), (0, D)),
    )
    grep = _kscan(memt, t, idx, tailt)
    acc = _k3_segment_sums(r, val, z)
    s, g = _k4_gather_sums(acc, grep, r)
    out = pl.pallas_call(
        _k5_body,
        grid=(B // _BLK,),
        in_specs=[
            pl.BlockSpec((_BLK, 2 * D), lambda i: (i, 0)),
            pl.BlockSpec((_BLK, D), lambda i: (i, 0)),
            pl.BlockSpec((D, D), lambda i: (0, 0)),
            pl.BlockSpec((1, D), lambda i: (0, 0)),
        ],
        out_specs=pl.BlockSpec((_BLK, D), lambda i: (i, 0)),
        out_shape=jax.ShapeDtypeStruct((B, D), jnp.float32),
    )(g, s, W, b.reshape(1, D))
    return out
